# baseline (speedup 1.0000x reference)
import functools

import jax
import jax.numpy as jnp
from jax import lax
from jax.experimental import pallas as pl
from jax.experimental.pallas import tpu as pltpu
from jax.experimental.pallas import tpu_sc as plsc

_LANES = 16
_CHUNK = 512
_K = 16


@functools.lru_cache(maxsize=None)
def _make_gather(V: int, B: int):
    info = plsc.get_sparse_core_info()
    nw = info.num_cores * info.num_subcores
    b_per_w = B // nw
    n_idx = b_per_w * _K
    n_chunks = n_idx // _CHUNK
    assert B % (8 * nw) == 0 and b_per_w % _CHUNK == 0
    mesh = plsc.VectorSubcoreMesh(core_axis_name="c", subcore_axis_name="s")

    @functools.partial(
        pl.kernel,
        mesh=mesh,
        compiler_params=pltpu.CompilerParams(
            use_tc_tiling_on_sc=False,
            disable_bounds_checks=True,
            disable_semaphore_checks=True,
            skip_device_barrier=True,
        ),
        out_type=jax.ShapeDtypeStruct((_K, B), jnp.float32),
        scratch_types=[
            pltpu.VMEM((b_per_w,), jnp.int32),
            pltpu.VMEM((n_idx,), jnp.int32),
            pltpu.VMEM((_K, b_per_w), jnp.float32),
            pltpu.SemaphoreType.DMA,
        ],
    )
    def gather_kernel(ids_hbm, table_hbm, out_hbm, idx_v, lst_v, rows_v, sem):
        wid = lax.axis_index("s") * info.num_cores + lax.axis_index("c")
        base = wid * b_per_w
        pltpu.sync_copy(ids_hbm.at[pl.ds(base, b_per_w)], idx_v)
        hi = jnp.full((_LANES,), V - 1, dtype=jnp.int32)
        lo = jnp.zeros((_LANES,), dtype=jnp.int32)
        for g in range(b_per_w // _LANES):
            sl = pl.ds(g * _LANES, _LANES)
            ids = jnp.minimum(jnp.maximum(idx_v[sl], lo), hi)
            for k in range(_K):
                lst_v[pl.ds(k * b_per_w + g * _LANES, _LANES)] = ids + k * V
        cpk = _CHUNK // b_per_w if _CHUNK > b_per_w else 0
        del cpk
        per_k = b_per_w // _CHUNK
        gathers = [
            pltpu.make_async_copy(
                table_hbm.at[lst_v.at[pl.ds(m * _CHUNK, _CHUNK)]],
                rows_v.at[m // per_k, pl.ds((m % per_k) * _CHUNK, _CHUNK)],
                sem,
            )
            for m in range(n_chunks)
        ]
        for c in gathers:
            c.start()
        for c in gathers:
            c.wait()
        pltpu.sync_copy(rows_v, out_hbm.at[:, pl.ds(base, b_per_w)])

    return gather_kernel


def kernel(camera_ids, base_poses):
    v = base_poses.shape[0]
    b = camera_ids.shape[0]
    table_t = jnp.transpose(base_poses, (1, 2, 0)).reshape(-1)
    out_t = _make_gather(v, b)(camera_ids.astype(jnp.int32), table_t)
    return jnp.transpose(out_t.reshape(4, 4, b), (2, 0, 1))


# speedup vs baseline: 1.0119x; 1.0119x over previous
import functools

import jax
import jax.numpy as jnp
from jax import lax
from jax.experimental import pallas as pl
from jax.experimental.pallas import tpu as pltpu
from jax.experimental.pallas import tpu_sc as plsc

_LANES = 16
_CHUNK = 512
_K = 16


@functools.lru_cache(maxsize=None)
def _make_gather(V: int, B: int):
    info = plsc.get_sparse_core_info()
    nw = info.num_cores * info.num_subcores
    b_per_w = B // nw
    n_idx = b_per_w * _K
    n_chunks = n_idx // _CHUNK
    assert B % (8 * nw) == 0 and b_per_w % _CHUNK == 0
    mesh = plsc.VectorSubcoreMesh(core_axis_name="c", subcore_axis_name="s")

    @functools.partial(
        pl.kernel,
        mesh=mesh,
        compiler_params=pltpu.CompilerParams(use_tc_tiling_on_sc=False),
        out_type=jax.ShapeDtypeStruct((_K, B), jnp.float32),
        scratch_types=[
            pltpu.VMEM((b_per_w,), jnp.int32),
            pltpu.VMEM((n_idx,), jnp.int32),
            pltpu.VMEM((_K, b_per_w), jnp.float32),
            pltpu.SemaphoreType.DMA((_K,)),
            pltpu.SemaphoreType.DMA,
        ],
    )
    def gather_kernel(ids_hbm, table_hbm, out_hbm, idx_v, lst_v, rows_v, gsem,
                      ssem):
        wid = lax.axis_index("s") * info.num_cores + lax.axis_index("c")
        base = wid * b_per_w
        pltpu.sync_copy(ids_hbm.at[pl.ds(base, b_per_w)], idx_v)
        hi = jnp.full((_LANES,), V - 1, dtype=jnp.int32)
        lo = jnp.zeros((_LANES,), dtype=jnp.int32)
        for g in range(b_per_w // _LANES):
            sl = pl.ds(g * _LANES, _LANES)
            ids = jnp.minimum(jnp.maximum(idx_v[sl], lo), hi)
            for k in range(_K):
                lst_v[pl.ds(k * b_per_w + g * _LANES, _LANES)] = ids + k * V
        gathers = [
            pltpu.make_async_copy(
                table_hbm.at[lst_v.at[pl.ds(m * _CHUNK, _CHUNK)]],
                rows_v.at[m, pl.ds(0, _CHUNK)],
                gsem.at[m],
            )
            for m in range(n_chunks)
        ]
        for c in gathers:
            c.start()
        stores = [
            pltpu.make_async_copy(
                rows_v.at[m],
                out_hbm.at[m, pl.ds(base, b_per_w)],
                ssem,
            )
            for m in range(n_chunks)
        ]
        for m in range(n_chunks):
            gathers[m].wait()
            stores[m].start()
        for c in stores:
            c.wait()

    return gather_kernel


def kernel(camera_ids, base_poses):
    v = base_poses.shape[0]
    b = camera_ids.shape[0]
    table_t = jnp.transpose(base_poses, (1, 2, 0)).reshape(-1)
    out_t = _make_gather(v, b)(camera_ids.astype(jnp.int32), table_t)
    return jnp.transpose(out_t.reshape(4, 4, b), (2, 0, 1))


# speedup vs baseline: 1.0157x; 1.0037x over previous
import functools

import jax
import jax.numpy as jnp
from jax import lax
from jax.experimental import pallas as pl
from jax.experimental.pallas import tpu as pltpu
from jax.experimental.pallas import tpu_sc as plsc

_LANES = 16
_CHUNK = 512
_K = 16


@functools.lru_cache(maxsize=None)
def _make_gather(V: int, B: int):
    info = plsc.get_sparse_core_info()
    nw = info.num_cores * info.num_subcores
    b_per_w = B // nw
    n_idx = b_per_w * _K
    n_chunks = n_idx // _CHUNK
    assert B % (8 * nw) == 0 and b_per_w == _CHUNK and n_chunks == _K
    mesh = plsc.VectorSubcoreMesh(core_axis_name="c", subcore_axis_name="s")

    @functools.partial(
        pl.kernel,
        mesh=mesh,
        compiler_params=pltpu.CompilerParams(use_tc_tiling_on_sc=False),
        out_type=jax.ShapeDtypeStruct((_K, B), jnp.float32),
        scratch_types=[
            pltpu.VMEM((b_per_w,), jnp.int32),
            pltpu.VMEM((n_idx,), jnp.int32),
            pltpu.VMEM((_K, b_per_w), jnp.float32),
            pltpu.SemaphoreType.DMA((_K,)),
            pltpu.SemaphoreType.DMA,
        ],
    )
    def gather_kernel(ids_hbm, table_hbm, out_hbm, idx_v, lst_v, rows_v, gsem,
                      ssem):
        wid = lax.axis_index("s") * info.num_cores + lax.axis_index("c")
        base = wid * b_per_w
        pltpu.sync_copy(ids_hbm.at[pl.ds(base, b_per_w)], idx_v)
        hi = jnp.full((_LANES,), V - 1, dtype=jnp.int32)
        lo = jnp.zeros((_LANES,), dtype=jnp.int32)
        for g in range(b_per_w // _LANES):
            sl = pl.ds(g * _LANES, _LANES)
            ids = jnp.minimum(jnp.maximum(idx_v[sl], lo), hi)
            for k in range(_K):
                lst_v[pl.ds(k * b_per_w + g * _LANES, _LANES)] = ids + k * V
        gathers = [
            pltpu.make_async_copy(
                table_hbm.at[lst_v.at[pl.ds(m * _CHUNK, _CHUNK)]],
                rows_v.at[m, pl.ds(0, _CHUNK)],
                gsem.at[m],
            )
            for m in range(n_chunks)
        ]
        for c in gathers:
            c.start()
        stores = [
            pltpu.make_async_copy(
                rows_v.at[m],
                out_hbm.at[m, pl.ds(base, b_per_w)],
                ssem,
            )
            for m in range(n_chunks)
        ]
        for m in range(n_chunks):
            gathers[m].wait()
            stores[m].start()
        for c in stores:
            c.wait()

    return gather_kernel


def kernel(camera_ids, base_poses):
    v = base_poses.shape[0]
    b = camera_ids.shape[0]
    table_t = jnp.transpose(base_poses, (1, 2, 0)).reshape(-1)
    out_t = _make_gather(v, b)(camera_ids.astype(jnp.int32), table_t)
    return jnp.transpose(out_t.reshape(4, 4, b), (2, 0, 1))
